# baseline (speedup 1.0000x reference)
import functools

import jax
import jax.numpy as jnp
from jax.experimental import pallas as pl
from jax.experimental.pallas import tpu as pltpu

_EPS = 1e-5


def _body(x_ref, e_ref,
          w1p_ref, b1p_ref, g1p_ref, be1p_ref, ap_ref,
          w2p_ref, b2p_ref, g2p_ref, be2p_ref,
          w1e_ref, b1e_ref, g1e_ref, be1e_ref, ae_ref,
          w2e_ref, b2e_ref, g2e_ref, be2e_ref,
          out_ref,
          s1p_ref, q1p_ref, s1e_ref, q1e_ref,
          sc1p_ref, sh1p_ref, sc1e_ref, sh1e_ref,
          s2p_ref, q2p_ref, s2e_ref, q2e_ref,
          sc2p_ref, sh2p_ref, sc2e_ref, sh2e_ref,
          *, inv_n):
    phase = pl.program_id(0)
    i = pl.program_id(1)
    x = x_ref[...]
    ev = e_ref[...]

    def z1_p():
        return jnp.dot(x, w1p_ref[...], preferred_element_type=jnp.float32) + b1p_ref[...]

    def z1_e():
        return jnp.dot(ev, w1e_ref[...], preferred_element_type=jnp.float32) + b1e_ref[...]

    @pl.when(phase == 0)
    def _():
        zp = z1_p()
        ze = z1_e()

        @pl.when(i == 0)
        def _():
            s1p_ref[...] = jnp.zeros_like(s1p_ref)
            q1p_ref[...] = jnp.zeros_like(q1p_ref)
            s1e_ref[...] = jnp.zeros_like(s1e_ref)
            q1e_ref[...] = jnp.zeros_like(q1e_ref)

        s1p_ref[...] += jnp.sum(zp, axis=0, keepdims=True)
        q1p_ref[...] += jnp.sum(zp * zp, axis=0, keepdims=True)
        s1e_ref[...] += jnp.sum(ze, axis=0, keepdims=True)
        q1e_ref[...] += jnp.sum(ze * ze, axis=0, keepdims=True)

    @pl.when((phase == 1) & (i == 0))
    def _():
        m = s1p_ref[...] * inv_n
        v = q1p_ref[...] * inv_n - m * m
        a = g1p_ref[...] * jax.lax.rsqrt(v + _EPS)
        sc1p_ref[...] = a
        sh1p_ref[...] = be1p_ref[...] - m * a
        m = s1e_ref[...] * inv_n
        v = q1e_ref[...] * inv_n - m * m
        a = g1e_ref[...] * jax.lax.rsqrt(v + _EPS)
        sc1e_ref[...] = a
        sh1e_ref[...] = be1e_ref[...] - m * a

    @pl.when(phase > 0)
    def _():
        yp = z1_p() * sc1p_ref[...] + sh1p_ref[...]
        p = jnp.where(yp >= 0, yp, ap_ref[0, 0] * yp)
        z2p = jnp.dot(p, w2p_ref[...], preferred_element_type=jnp.float32) + b2p_ref[...]

        ye = z1_e() * sc1e_ref[...] + sh1e_ref[...]
        pe = jnp.where(ye >= 0, ye, ae_ref[0, 0] * ye)
        z2e = jnp.dot(pe, w2e_ref[...], preferred_element_type=jnp.float32) + b2e_ref[...]

        @pl.when(phase == 1)
        def _():
            @pl.when(i == 0)
            def _():
                s2p_ref[...] = jnp.zeros_like(s2p_ref)
                q2p_ref[...] = jnp.zeros_like(q2p_ref)
                s2e_ref[...] = jnp.zeros_like(s2e_ref)
                q2e_ref[...] = jnp.zeros_like(q2e_ref)

            s2p_ref[...] += jnp.sum(z2p, axis=0, keepdims=True)
            q2p_ref[...] += jnp.sum(z2p * z2p, axis=0, keepdims=True)
            s2e_ref[...] += jnp.sum(z2e, axis=0, keepdims=True)
            q2e_ref[...] += jnp.sum(z2e * z2e, axis=0, keepdims=True)

        @pl.when(phase == 2)
        def _():
            @pl.when(i == 0)
            def _():
                m = s2p_ref[...] * inv_n
                v = q2p_ref[...] * inv_n - m * m
                a = g2p_ref[...] * jax.lax.rsqrt(v + _EPS)
                sc2p_ref[...] = a
                sh2p_ref[...] = be2p_ref[...] - m * a
                m = s2e_ref[...] * inv_n
                v = q2e_ref[...] * inv_n - m * m
                a = g2e_ref[...] * jax.lax.rsqrt(v + _EPS)
                sc2e_ref[...] = a
                sh2e_ref[...] = be2e_ref[...] - m * a

            op = z2p * sc2p_ref[...] + sh2p_ref[...]
            oe = z2e * sc2e_ref[...] + sh2e_ref[...]
            out_ref[...] = jnp.concatenate([op, oe], axis=-1)


def kernel(last, extra, W1p, b1p, g1p, be1p, a1p, W2p, b2p, g2p, be2p,
           W1e, b1e, g1e, be1e, a1e, W2e, b2e, g2e, be2e):
    n = last.shape[0]
    blk = 2000
    nb = n // blk
    assert nb * blk == n

    def row(v):
        return v.reshape(1, -1)

    args = (last, extra,
            W1p.T, row(b1p), row(g1p), row(be1p), a1p.reshape(1, 1),
            W2p.T, row(b2p), row(g2p), row(be2p),
            W1e.T, row(b1e), row(g1e), row(be1e), a1e.reshape(1, 1),
            W2e.T, row(b2e), row(g2e), row(be2e))

    def big(d):
        return pl.BlockSpec((blk, d), lambda ph, i: (i, 0))

    def full(shape):
        return pl.BlockSpec(shape, lambda ph, i: (0, 0))

    in_specs = [
        big(3), big(16),
        full((3, 192)), full((1, 192)), full((1, 192)), full((1, 192)), full((1, 1)),
        full((192, 96)), full((1, 96)), full((1, 96)), full((1, 96)),
        full((16, 64)), full((1, 64)), full((1, 64)), full((1, 64)), full((1, 1)),
        full((64, 32)), full((1, 32)), full((1, 32)), full((1, 32)),
    ]
    out_spec = pl.BlockSpec((blk, 128), lambda ph, i: ((ph // 2) * i, 0))

    scratch = [pltpu.VMEM((1, d), jnp.float32)
               for d in (192, 192, 64, 64,
                         192, 192, 64, 64,
                         96, 96, 32, 32,
                         96, 96, 32, 32)]

    return pl.pallas_call(
        functools.partial(_body, inv_n=1.0 / n),
        grid=(3, nb),
        in_specs=in_specs,
        out_specs=out_spec,
        out_shape=jax.ShapeDtypeStruct((n, 128), jnp.float32),
        scratch_shapes=scratch,
        compiler_params=pltpu.CompilerParams(
            dimension_semantics=("arbitrary", "arbitrary")),
    )(*args)
